# Initial kernel scaffold; baseline (speedup 1.0000x reference)
import jax
import jax.numpy as jnp
from jax import lax
from jax.experimental import pallas as pl
from jax.experimental.pallas import tpu as pltpu
from jax.experimental.pallas import tpu_sc as plsc

N = 10000
D = 128
E = 320000
NT = 32
NSUB = 16
B = 128
K = -(-E // (NT * B))
EPAD = NT * K * B
NPAD = 10240
DEG_PT = NPAD // NSUB
ROWS_PT = N // NSUB

_mesh = plsc.VectorSubcoreMesh(core_axis_name="c", subcore_axis_name="s")


def _deg_body(dst_r, w_r, zeros, out, idx_d, wv, acc):
    cid = lax.axis_index("c")
    sid = lax.axis_index("s")
    wid = cid * NSUB + sid
    pltpu.sync_copy(dst_r.at[wid], idx_d)
    pltpu.sync_copy(w_r.at[wid], wv)
    pltpu.sync_copy(zeros.at[pl.ds(sid * DEG_PT, DEG_PT)],
                    acc.at[pl.ds(sid * DEG_PT, DEG_PT)])
    plsc.subcore_barrier()

    def step(j, c):
        pltpu.sync_copy(wv.at[j], acc.at[idx_d.at[j]], add=True)
        return c

    lax.fori_loop(0, K, step, 0)
    plsc.subcore_barrier()
    pltpu.sync_copy(acc.at[pl.ds(sid * DEG_PT, DEG_PT)],
                    out.at[cid, pl.ds(sid * DEG_PT, DEG_PT)])


_deg_call = pl.kernel(
    _deg_body,
    out_type=jax.ShapeDtypeStruct((2, NPAD), jnp.float32),
    mesh=_mesh,
    scratch_types=[
        pltpu.VMEM((K, B), jnp.int32),
        pltpu.VMEM((K, B), jnp.float32),
        pltpu.VMEM_SHARED((NPAD,), jnp.float32),
    ],
)


def _agg_body(g, src_r, dst_r, w_r, zrows, out, idx_s, idx_d, wv, rows, acc, sem):
    cid = lax.axis_index("c")
    sid = lax.axis_index("s")
    wid = cid * NSUB + sid
    pltpu.sync_copy(src_r.at[wid], idx_s)
    pltpu.sync_copy(dst_r.at[wid], idx_d)
    pltpu.sync_copy(w_r.at[wid], wv)
    pltpu.sync_copy(zrows.at[pl.ds(sid * ROWS_PT, ROWS_PT)],
                    acc.at[pl.ds(sid * ROWS_PT, ROWS_PT)])
    plsc.subcore_barrier()

    def step(j, c):
        pltpu.async_copy(g.at[idx_s.at[j]], rows, sem).wait()

        def scale(e, c2):
            we = wv[j, e]
            for dd in range(D // 16):
                sl = pl.ds(dd * 16, 16)
                rows[e, sl] = rows[e, sl] * we
            return c2

        lax.fori_loop(0, B, scale, 0)
        pltpu.sync_copy(rows, acc.at[idx_d.at[j]], add=True)
        return c

    lax.fori_loop(0, K, step, 0)
    plsc.subcore_barrier()
    pltpu.sync_copy(acc.at[pl.ds(sid * ROWS_PT, ROWS_PT)],
                    out.at[cid, pl.ds(sid * ROWS_PT, ROWS_PT)])


_agg_call = pl.kernel(
    _agg_body,
    out_type=jax.ShapeDtypeStruct((2, N, D), jnp.float32),
    mesh=_mesh,
    scratch_types=[
        pltpu.VMEM((K, B), jnp.int32),
        pltpu.VMEM((K, B), jnp.int32),
        pltpu.VMEM((K, B), jnp.float32),
        pltpu.VMEM((B, D), jnp.float32),
        pltpu.VMEM_SHARED((N, D), jnp.float32),
        pltpu.SemaphoreType.DMA,
    ],
)


def _dis_body(p_ref, dis_ref):
    dis_ref[...] = lax.rsqrt(1.0 + p_ref[0] + p_ref[1])


_dis_call = pl.pallas_call(
    _dis_body,
    out_shape=jax.ShapeDtypeStruct((NPAD // 128, 128), jnp.float32),
)


def _first_body(x_ref, w_ref, dis_ref, g_ref):
    h = jnp.dot(x_ref[...], w_ref[...], preferred_element_type=jnp.float32)
    g_ref[...] = h * dis_ref[...]


_first_call = pl.pallas_call(
    _first_body,
    out_shape=jax.ShapeDtypeStruct((N, D), jnp.float32),
)


def _mid_body(p0_ref, p1_ref, g_ref, dis_ref, b_ref, w_ref, gout_ref):
    s = dis_ref[...] * (p0_ref[...] + p1_ref[...] + g_ref[...]) + b_ref[...]
    a = jnp.maximum(s, 0.0)
    h = jnp.dot(a, w_ref[...], preferred_element_type=jnp.float32)
    gout_ref[...] = h * dis_ref[...]


_mid_call = pl.pallas_call(
    _mid_body,
    out_shape=jax.ShapeDtypeStruct((N, D), jnp.float32),
)


def _final_body(p0_ref, p1_ref, g_ref, dis_ref, b_ref, out_ref):
    out_ref[...] = dis_ref[...] * (p0_ref[...] + p1_ref[...] + g_ref[...]) + b_ref[...]


_final_call = pl.pallas_call(
    _final_body,
    out_shape=jax.ShapeDtypeStruct((N, D), jnp.float32),
)


def kernel(x, edge_index, edge_weight, W1, b1, W2, b2, W3, b3):
    src = edge_index[0]
    dst = edge_index[1]
    pad = EPAD - E
    zi = jnp.zeros((pad,), jnp.int32)
    src_r = jnp.concatenate([src, zi]).reshape(NT, K, B)
    dst_r = jnp.concatenate([dst, zi]).reshape(NT, K, B)
    w_r = jnp.concatenate([edge_weight, jnp.zeros((pad,), jnp.float32)]).reshape(NT, K, B)
    zero_deg = jnp.zeros((NPAD,), jnp.float32)
    zero_rows = jnp.zeros((N, D), jnp.float32)

    degp = _deg_call(dst_r, w_r, zero_deg)
    dis2d = _dis_call(degp.reshape(2, NPAD // 128, 128))
    dis_col = dis2d.reshape(NPAD, 1)[:N]

    g = _first_call(x, W1, dis_col)
    p = _agg_call(g, src_r, dst_r, w_r, zero_rows)
    g = _mid_call(p[0], p[1], g, dis_col, b1.reshape(1, D), W2)
    p = _agg_call(g, src_r, dst_r, w_r, zero_rows)
    g = _mid_call(p[0], p[1], g, dis_col, b2.reshape(1, D), W3)
    p = _agg_call(g, src_r, dst_r, w_r, zero_rows)
    out = _final_call(p[0], p[1], g, dis_col, b3.reshape(1, D))
    return out


# speedup vs baseline: 9.5519x; 9.5519x over previous
import jax
import jax.numpy as jnp
from jax import lax
from jax.experimental import pallas as pl
from jax.experimental.pallas import tpu as pltpu
from jax.experimental.pallas import tpu_sc as plsc

N = 10000
D = 128
E = 320000
NT = 32
NSUB = 16
B = 128
K = -(-E // (NT * B))
EPAD = NT * K * B
NPAD = 10240
DEG_PT = NPAD // NSUB
ROWS_PT = NPAD // NSUB

_mesh = plsc.VectorSubcoreMesh(core_axis_name="c", subcore_axis_name="s")


def _deg_body(dst_r, w_r, zeros, out, idx_d, wv, acc):
    cid = lax.axis_index("c")
    sid = lax.axis_index("s")
    wid = cid * NSUB + sid
    pltpu.sync_copy(dst_r.at[wid], idx_d)
    pltpu.sync_copy(w_r.at[wid], wv)
    pltpu.sync_copy(zeros.at[pl.ds(sid * DEG_PT, DEG_PT)],
                    acc.at[pl.ds(sid * DEG_PT, DEG_PT)])
    plsc.subcore_barrier()

    def step(j, c):
        pltpu.sync_copy(wv.at[j], acc.at[idx_d.at[j]], add=True)
        return c

    lax.fori_loop(0, K, step, 0)
    plsc.subcore_barrier()
    pltpu.sync_copy(acc.at[pl.ds(sid * DEG_PT, DEG_PT)],
                    out.at[cid, pl.ds(sid * DEG_PT, DEG_PT)])


_deg_call = pl.kernel(
    _deg_body,
    out_type=jax.ShapeDtypeStruct((2, NPAD), jnp.float32),
    mesh=_mesh,
    scratch_types=[
        pltpu.VMEM((K, B), jnp.int32),
        pltpu.VMEM((K, B), jnp.float32),
        pltpu.VMEM_SHARED((NPAD,), jnp.float32),
    ],
)


def _agg_body(g, src_r, dst_r, w_r, zrows, out, idx_s, idx_d, wv, rows, acc, sem):
    cid = lax.axis_index("c")
    sid = lax.axis_index("s")
    wid = cid * NSUB + sid
    pltpu.sync_copy(src_r.at[wid], idx_s)
    pltpu.sync_copy(dst_r.at[wid], idx_d)
    pltpu.sync_copy(w_r.at[wid], wv)
    pltpu.sync_copy(zrows.at[pl.ds(sid * ROWS_PT, ROWS_PT)],
                    acc.at[pl.ds(sid * ROWS_PT, ROWS_PT)])
    plsc.subcore_barrier()

    def step(j, c):
        pltpu.async_copy(g.at[idx_s.at[j]], rows, sem).wait()

        def scale16(q, c2):
            base = q * 16
            wchunk = wv[j, pl.ds(base, 16)]
            for e in range(16):
                we = wchunk[e]
                for dd in range(D // 16):
                    sl = pl.ds(dd * 16, 16)
                    rows[base + e, sl] = rows[base + e, sl] * we
            return c2

        lax.fori_loop(0, B // 16, scale16, 0)
        pltpu.sync_copy(rows, acc.at[idx_d.at[j]], add=True)
        return c

    lax.fori_loop(0, K, step, 0)
    plsc.subcore_barrier()
    pltpu.sync_copy(acc.at[pl.ds(sid * ROWS_PT, ROWS_PT)],
                    out.at[cid, pl.ds(sid * ROWS_PT, ROWS_PT)])


_agg_call = pl.kernel(
    _agg_body,
    out_type=jax.ShapeDtypeStruct((2, NPAD, D), jnp.float32),
    mesh=_mesh,
    scratch_types=[
        pltpu.VMEM((K, B), jnp.int32),
        pltpu.VMEM((K, B), jnp.int32),
        pltpu.VMEM((K, B), jnp.float32),
        pltpu.VMEM((B, D), jnp.float32),
        pltpu.VMEM_SHARED((NPAD, D), jnp.float32),
        pltpu.SemaphoreType.DMA,
    ],
)


def _dis_body(p_ref, dis_ref):
    dis_ref[...] = lax.rsqrt(1.0 + p_ref[0] + p_ref[1])


_dis_call = pl.pallas_call(
    _dis_body,
    out_shape=jax.ShapeDtypeStruct((NPAD // 128, 128), jnp.float32),
)


def _first_body(x_ref, w_ref, dis_ref, g_ref):
    h = jnp.dot(x_ref[...], w_ref[...], preferred_element_type=jnp.float32)
    g_ref[...] = h * dis_ref[...]


_first_call = pl.pallas_call(
    _first_body,
    out_shape=jax.ShapeDtypeStruct((N, D), jnp.float32),
)


def _mid_body(p0_ref, p1_ref, g_ref, dis_ref, b_ref, w_ref, gout_ref):
    s = dis_ref[...] * (p0_ref[...] + p1_ref[...] + g_ref[...]) + b_ref[...]
    a = jnp.maximum(s, 0.0)
    h = jnp.dot(a, w_ref[...], preferred_element_type=jnp.float32)
    gout_ref[...] = h * dis_ref[...]


_mid_call = pl.pallas_call(
    _mid_body,
    out_shape=jax.ShapeDtypeStruct((N, D), jnp.float32),
)


def _final_body(p0_ref, p1_ref, g_ref, dis_ref, b_ref, out_ref):
    out_ref[...] = dis_ref[...] * (p0_ref[...] + p1_ref[...] + g_ref[...]) + b_ref[...]


_final_call = pl.pallas_call(
    _final_body,
    out_shape=jax.ShapeDtypeStruct((N, D), jnp.float32),
)


def kernel(x, edge_index, edge_weight, W1, b1, W2, b2, W3, b3):
    src = edge_index[0]
    dst = edge_index[1]
    pad = EPAD - E
    zi = jnp.zeros((pad,), jnp.int32)
    src_r = jnp.concatenate([src, zi]).reshape(NT, K, B)
    dst_r = jnp.concatenate([dst, zi]).reshape(NT, K, B)
    w_r = jnp.concatenate([edge_weight, jnp.zeros((pad,), jnp.float32)]).reshape(NT, K, B)
    zero_deg = jnp.zeros((NPAD,), jnp.float32)
    zero_rows = jnp.zeros((NPAD, D), jnp.float32)

    degp = _deg_call(dst_r, w_r, zero_deg)
    dis2d = _dis_call(degp.reshape(2, NPAD // 128, 128))
    dis_col = dis2d.reshape(NPAD, 1)[:N]

    g = _first_call(x, W1, dis_col)
    p = _agg_call(g, src_r, dst_r, w_r, zero_rows)
    g = _mid_call(p[0, :N], p[1, :N], g, dis_col, b1.reshape(1, D), W2)
    p = _agg_call(g, src_r, dst_r, w_r, zero_rows)
    g = _mid_call(p[0, :N], p[1, :N], g, dis_col, b2.reshape(1, D), W3)
    p = _agg_call(g, src_r, dst_r, w_r, zero_rows)
    out = _final_call(p[0, :N], p[1, :N], g, dis_col, b3.reshape(1, D))
    return out
